# baseline (speedup 1.0000x reference)
import functools

import jax
import jax.numpy as jnp
from jax import lax
from jax.experimental import pallas as pl
from jax.experimental.pallas import tpu as pltpu
from jax.experimental.pallas import tpu_sc as plsc

N = 10000
M = 10
D_FEAT = 256
EMBED = 128
T = 3

NC = 2
NS = 16
NW = NC * NS
LANES = 16

U = 12
UM = U * M
SP = 128
DEPTH = 2
UPW = 236
CPW = U * UPW
R = T * T * N
RP = NW * CPW


_BP = 2000


def _proj_body(cf, df, gf, wp, bp, out):
    for t, f in enumerate((cf, df, gf)):
        acc = jnp.dot(f[...], wp[t], preferred_element_type=jnp.float32)
        out[t] = jax.nn.relu(acc + bp[t])


def _project(cell_feat, drug_feat, gene_feat, W_proj, b_proj):
    grid = (N // _BP,)
    fspec = pl.BlockSpec((_BP, D_FEAT), lambda i: (i, 0))
    return pl.pallas_call(
        _proj_body,
        grid=grid,
        in_specs=[
            fspec, fspec, fspec,
            pl.BlockSpec((T, D_FEAT, EMBED), lambda i: (0, 0, 0)),
            pl.BlockSpec((T, EMBED), lambda i: (0, 0)),
        ],
        out_specs=pl.BlockSpec((T, _BP, EMBED), lambda i: (0, i, 0)),
        out_shape=jax.ShapeDtypeStruct((T, N, EMBED), jnp.float32),
    )(cell_feat, drug_feat, gene_feat, W_proj, b_proj)


def _agg_sc(h_flat, idx, wn):
    mesh = plsc.VectorSubcoreMesh(
        core_axis_name="c", subcore_axis_name="s", num_cores=NC, num_subcores=NS
    )

    @functools.partial(
        pl.kernel,
        out_type=jax.ShapeDtypeStruct((RP * EMBED,), jnp.float32),
        mesh=mesh,
        scratch_types=[
            pltpu.VMEM((UPW * SP,), jnp.int32),
            pltpu.VMEM((CPW * M + 8,), jnp.float32),
            *[pltpu.VMEM((UM, EMBED), jnp.float32) for _ in range(DEPTH)],
            *[pltpu.VMEM((U * EMBED,), jnp.float32) for _ in range(DEPTH)],
            *[pltpu.SemaphoreType.DMA for _ in range(2 * DEPTH + 1)],
        ],
    )
    def agg_kernel(h_hbm, idx_hbm, wn_hbm, out_hbm, idx_v, wn_v, *rest):
        gbufs = rest[:DEPTH]
        obufs = rest[DEPTH:2 * DEPTH]
        gsems = rest[2 * DEPTH:3 * DEPTH]
        osems = rest[3 * DEPTH:4 * DEPTH]
        bsem = rest[4 * DEPTH]
        wid = lax.axis_index("c") * NS + lax.axis_index("s")
        wbase = wid * CPW

        pltpu.async_copy(
            idx_hbm.at[pl.ds(wid * UPW * SP, UPW * SP)], idx_v, bsem).wait()
        pltpu.async_copy(
            wn_hbm.at[pl.ds(wbase * M, CPW * M)], wn_v.at[pl.ds(0, CPW * M)], bsem
        ).wait()

        def fire_gathers(r, u):
            pltpu.async_copy(h_hbm.at[idx_v.at[pl.ds(u * SP, UM)]],
                             gbufs[r], gsems[r])

        def drain_gathers(r, u):
            pltpu.make_async_copy(h_hbm.at[idx_v.at[pl.ds(u * SP, UM)]],
                                  gbufs[r], gsems[r]).wait()

        def odst(u):
            return out_hbm.at[pl.ds((wbase + u * U) * EMBED, U * EMBED)]

        for u in range(DEPTH - 1):
            fire_gathers(u, u)

        @pl.loop(0, UPW, step=DEPTH)
        def _(u0):
            for r in range(DEPTH):
                u = u0 + r
                rn = (r + DEPTH - 1) % DEPTH

                @pl.when(u + DEPTH - 1 < UPW)
                def _():
                    fire_gathers(rn, u + DEPTH - 1)

                drain_gathers(r, u)

                @pl.when(u >= DEPTH)
                def _():
                    pltpu.make_async_copy(obufs[r], odst(u), osems[r]).wait()

                @pl.loop(0, U)
                def _(b):
                    wvec = wn_v[pl.ds(u * UM + b * M, LANES)]
                    ws = [wvec[m] for m in range(M)]
                    den = ws[0]
                    for m in range(1, M):
                        den = den + ws[m]
                    rec = jnp.full((LANES,), 1.0, jnp.float32) / jnp.maximum(
                        den, 1e-6)
                    for c in range(EMBED // LANES):
                        sl = pl.ds(c * LANES, LANES)
                        acc = ws[0] * gbufs[r][b * M, sl]
                        for m in range(1, M):
                            acc = acc + ws[m] * gbufs[r][b * M + m, sl]
                        obufs[r][pl.ds(b * EMBED + c * LANES, LANES)] = acc * rec

                pltpu.async_copy(obufs[r], odst(u), osems[r])

        for r in range(DEPTH):
            pltpu.make_async_copy(
                obufs[r], odst(UPW - DEPTH + r), osems[r]).wait()

    return agg_kernel(h_flat, idx, wn)


_BC = 2000


def _comb_body(h_ref, a0, a1, a2, wg, bg, out):
    cat = jnp.concatenate([h_ref[0], a0[0, 0], a1[0, 0], a2[0, 0]], axis=-1)
    acc = jnp.dot(cat, wg[0], preferred_element_type=jnp.float32)
    out[0] = jax.nn.relu(acc + bg[pl.ds(pl.program_id(0), 1), :])


def _combine(h, agg, W_gnn, b_gnn):
    grid = (T, N // _BC)
    aspec = lambda nt: pl.BlockSpec(
        (1, 1, _BC, EMBED), lambda t, i, nt=nt: (t, nt, i, 0)
    )
    return pl.pallas_call(
        _comb_body,
        grid=grid,
        in_specs=[
            pl.BlockSpec((1, _BC, EMBED), lambda t, i: (t, i, 0)),
            aspec(0), aspec(1), aspec(2),
            pl.BlockSpec((1, (1 + T) * EMBED, EMBED), lambda t, i: (t, 0, 0)),
            pl.BlockSpec((T, EMBED), lambda t, i: (0, 0)),
        ],
        out_specs=pl.BlockSpec((1, _BC, EMBED), lambda t, i: (t, i, 0)),
        out_shape=jax.ShapeDtypeStruct((T, N, EMBED), jnp.float32),
    )(h, agg, agg, agg, W_gnn, b_gnn)


@jax.jit
def kernel(cell_feat, drug_feat, gene_feat, neighbor_lids, neighbor_weights,
           neighbor_masks, W_proj, b_proj, W_gnn, b_gnn):
    h = _project(cell_feat, drug_feat, gene_feat, W_proj, b_proj)
    h_flat = h.reshape(T * N, EMBED)

    offs = (jnp.arange(T, dtype=jnp.int32) * N)[None, :, None, None]
    gidx = (neighbor_lids + offs).reshape(R * M)
    gidx = jnp.pad(gidx, (0, (RP - R) * M)).reshape(RP // U, UM)
    gidx = jnp.pad(gidx, ((0, 0), (0, SP - UM))).reshape(RP // U * SP)

    wm = (neighbor_weights * neighbor_masks.astype(jnp.float32)).reshape(R * M)
    wm = jnp.pad(wm, (0, (RP - R) * M))

    agg = _agg_sc(h_flat, gidx, wm)[:R * EMBED].reshape(T, T, N, EMBED)
    return _combine(h, agg, W_gnn, b_gnn)


# speedup vs baseline: 1.0148x; 1.0148x over previous
import functools

import jax
import jax.numpy as jnp
from jax import lax
from jax.experimental import pallas as pl
from jax.experimental.pallas import tpu as pltpu
from jax.experimental.pallas import tpu_sc as plsc

N = 10000
M = 10
D_FEAT = 256
EMBED = 128
T = 3

NC = 2
NS = 16
NW = NC * NS
LANES = 16

U = 12
UM = U * M
SP = 128
DEPTH = 2
UPW = 236
CPW = U * UPW
R = T * T * N
RP = NW * CPW


_BP = 2000


def _proj_body(cf, df, gf, wp, bp, out):
    for t, f in enumerate((cf, df, gf)):
        acc = jnp.dot(f[...], wp[t], preferred_element_type=jnp.float32)
        out[t] = jax.nn.relu(acc + bp[t])


def _project(cell_feat, drug_feat, gene_feat, W_proj, b_proj):
    grid = (N // _BP,)
    fspec = pl.BlockSpec((_BP, D_FEAT), lambda i: (i, 0))
    return pl.pallas_call(
        _proj_body,
        grid=grid,
        in_specs=[
            fspec, fspec, fspec,
            pl.BlockSpec((T, D_FEAT, EMBED), lambda i: (0, 0, 0)),
            pl.BlockSpec((T, EMBED), lambda i: (0, 0)),
        ],
        out_specs=pl.BlockSpec((T, _BP, EMBED), lambda i: (0, i, 0)),
        out_shape=jax.ShapeDtypeStruct((T, N, EMBED), jnp.float32),
    )(cell_feat, drug_feat, gene_feat, W_proj, b_proj)


_BW = 2000


def _wnorm_body(wm_ref, wn_ref):
    wm = wm_ref[...]
    den = jnp.clip(jnp.sum(wm, axis=1, keepdims=True), 1e-6, None)
    wn_ref[...] = wm / den


def _wnorm(wm):
    grid = (R // _BW,)
    return pl.pallas_call(
        _wnorm_body,
        grid=grid,
        in_specs=[pl.BlockSpec((_BW, M), lambda i: (i, 0))],
        out_specs=pl.BlockSpec((_BW, M), lambda i: (i, 0)),
        out_shape=jax.ShapeDtypeStruct((R, M), jnp.float32),
    )(wm)


def _agg_sc(h_flat, idx, wn):
    mesh = plsc.VectorSubcoreMesh(
        core_axis_name="c", subcore_axis_name="s", num_cores=NC, num_subcores=NS
    )

    @functools.partial(
        pl.kernel,
        out_type=jax.ShapeDtypeStruct((RP * EMBED,), jnp.float32),
        mesh=mesh,
        scratch_types=[
            pltpu.VMEM((UPW * SP,), jnp.int32),
            pltpu.VMEM((CPW * M + 8,), jnp.float32),
            *[pltpu.VMEM((UM, EMBED), jnp.float32) for _ in range(DEPTH)],
            *[pltpu.VMEM((U * EMBED,), jnp.float32) for _ in range(DEPTH)],
            *[pltpu.SemaphoreType.DMA for _ in range(2 * DEPTH + 1)],
        ],
    )
    def agg_kernel(h_hbm, idx_hbm, wn_hbm, out_hbm, idx_v, wn_v, *rest):
        gbufs = rest[:DEPTH]
        obufs = rest[DEPTH:2 * DEPTH]
        gsems = rest[2 * DEPTH:3 * DEPTH]
        osems = rest[3 * DEPTH:4 * DEPTH]
        bsem = rest[4 * DEPTH]
        wid = lax.axis_index("c") * NS + lax.axis_index("s")
        wbase = wid * CPW

        pltpu.async_copy(
            idx_hbm.at[pl.ds(wid * UPW * SP, UPW * SP)], idx_v, bsem).wait()
        pltpu.async_copy(
            wn_hbm.at[pl.ds(wbase * M, CPW * M)], wn_v.at[pl.ds(0, CPW * M)], bsem
        ).wait()

        def fire_gathers(r, u):
            pltpu.async_copy(h_hbm.at[idx_v.at[pl.ds(u * SP, UM)]],
                             gbufs[r], gsems[r])

        def drain_gathers(r, u):
            pltpu.make_async_copy(h_hbm.at[idx_v.at[pl.ds(u * SP, UM)]],
                                  gbufs[r], gsems[r]).wait()

        def odst(u):
            return out_hbm.at[pl.ds((wbase + u * U) * EMBED, U * EMBED)]

        for u in range(DEPTH - 1):
            fire_gathers(u, u)

        @pl.loop(0, UPW, step=DEPTH)
        def _(u0):
            for r in range(DEPTH):
                u = u0 + r
                rn = (r + DEPTH - 1) % DEPTH

                @pl.when(u + DEPTH - 1 < UPW)
                def _():
                    fire_gathers(rn, u + DEPTH - 1)

                drain_gathers(r, u)

                @pl.when(u >= DEPTH)
                def _():
                    pltpu.make_async_copy(obufs[r], odst(u), osems[r]).wait()

                @pl.loop(0, U)
                def _(b):
                    wvec = wn_v[pl.ds(u * UM + b * M, LANES)]
                    ws = [wvec[m] for m in range(M)]
                    for c in range(EMBED // LANES):
                        sl = pl.ds(c * LANES, LANES)
                        acc = ws[0] * gbufs[r][b * M, sl]
                        for m in range(1, M):
                            acc = acc + ws[m] * gbufs[r][b * M + m, sl]
                        obufs[r][pl.ds(b * EMBED + c * LANES, LANES)] = acc

                pltpu.async_copy(obufs[r], odst(u), osems[r])

        for r in range(DEPTH):
            pltpu.make_async_copy(
                obufs[r], odst(UPW - DEPTH + r), osems[r]).wait()

    return agg_kernel(h_flat, idx, wn)


_BC = 2000


def _comb_body(h_ref, a0, a1, a2, wg, bg, out):
    cat = jnp.concatenate([h_ref[0], a0[0, 0], a1[0, 0], a2[0, 0]], axis=-1)
    acc = jnp.dot(cat, wg[0], preferred_element_type=jnp.float32)
    out[0] = jax.nn.relu(acc + bg[pl.ds(pl.program_id(0), 1), :])


def _combine(h, agg, W_gnn, b_gnn):
    grid = (T, N // _BC)
    aspec = lambda nt: pl.BlockSpec(
        (1, 1, _BC, EMBED), lambda t, i, nt=nt: (t, nt, i, 0)
    )
    return pl.pallas_call(
        _comb_body,
        grid=grid,
        in_specs=[
            pl.BlockSpec((1, _BC, EMBED), lambda t, i: (t, i, 0)),
            aspec(0), aspec(1), aspec(2),
            pl.BlockSpec((1, (1 + T) * EMBED, EMBED), lambda t, i: (t, 0, 0)),
            pl.BlockSpec((T, EMBED), lambda t, i: (0, 0)),
        ],
        out_specs=pl.BlockSpec((1, _BC, EMBED), lambda t, i: (t, i, 0)),
        out_shape=jax.ShapeDtypeStruct((T, N, EMBED), jnp.float32),
    )(h, agg, agg, agg, W_gnn, b_gnn)


@jax.jit
def kernel(cell_feat, drug_feat, gene_feat, neighbor_lids, neighbor_weights,
           neighbor_masks, W_proj, b_proj, W_gnn, b_gnn):
    h = _project(cell_feat, drug_feat, gene_feat, W_proj, b_proj)
    h_flat = h.reshape(T * N, EMBED)

    offs = (jnp.arange(T, dtype=jnp.int32) * N)[None, :, None, None]
    gidx = (neighbor_lids + offs).reshape(R * M)
    gidx = jnp.pad(gidx, (0, (RP - R) * M)).reshape(RP // U, UM)
    gidx = jnp.pad(gidx, ((0, 0), (0, SP - UM))).reshape(RP // U * SP)

    wm = (neighbor_weights * neighbor_masks.astype(jnp.float32)).reshape(R, M)
    wn = _wnorm(wm).reshape(R * M)
    wn = jnp.pad(wn, (0, (RP - R) * M))

    agg = _agg_sc(h_flat, gidx, wn)[:R * EMBED].reshape(T, T, N, EMBED)
    return _combine(h, agg, W_gnn, b_gnn)
